# baseline (speedup 1.0000x reference)
import functools

import jax
import jax.numpy as jnp
import numpy as np
from jax import lax
from jax.experimental import pallas as pl
from jax.experimental.pallas import tpu as pltpu
from jax.experimental.pallas import tpu_sc as plsc

N = 100000
E = 1600000
D_IN = 32
D_HID = 32
D_OUT = 16

NC = 2
NS = 16

CH = 128
NCHUNK = 12544
EPAD = NCHUNK * CH
NPAD = 100096
RPT = NPAD // NS
NPAD2 = 102400
DZ = RPT // 2
TN = 25600 * 8

L1_PER_TILE = NCHUNK // NS
L2_PER_TILE = NCHUNK // (NC * NS)
KB = 8

_mesh = plsc.VectorSubcoreMesh(
    core_axis_name="c", subcore_axis_name="s", num_cores=NC, num_subcores=NS)

_sc_params = pltpu.CompilerParams(use_tc_tiling_on_sc=False)


def _transform_idx(sblk, slot, nrows, mult_off):
    off = mult_off
    for row in range(nrows):
        for i in range(CH // 16):
            v = sblk[slot, row, pl.ds(i * 16, 16)]
            sblk[slot, row, pl.ds(i * 16, 16)] = v * 2 + off


def _edge_pipeline(table, srcc, dstc, acc, t0, nchunks, idx_off,
                   sblk, dblk, rowsb, gsem, ssem, isem,
                   deg=None, ones=None, dsem=None, zdeg=None):
    gpb = KB // 4
    nblk = nchunks // KB
    ngroups = nchunks // 4

    def drain(dummy_src, dst, sem):
        pltpu.make_async_copy(dummy_src, dst, sem).wait()

    rows_dummy = table.at[pl.ds(0, CH)]

    pltpu.sync_copy(srcc.at[pl.ds(t0, KB)], sblk.at[0])
    pltpu.sync_copy(dstc.at[pl.ds(t0, KB)], dblk.at[0])
    _transform_idx(sblk, 0, KB, idx_off)
    for k in range(4):
        pltpu.async_copy(table.at[sblk.at[0, k]], rowsb.at[k], gsem.at[k])

    def group(g, p, b, rg):
        for k in range(4):
            q = p * 4 + k
            drain(rows_dummy, rowsb.at[q], gsem.at[q])
            pltpu.async_copy(rowsb.at[q], acc.at[dblk.at[b & 1, rg * 4 + k]],
                             ssem.at[q], add=True)
            if deg is not None:
                pltpu.async_copy(ones, deg.at[dblk.at[b & 1, rg * 4 + k]],
                                 dsem.at[q], add=True)

        slot2_b = (b + 1) & 1

        @pl.when((rg == gpb - 1) & (b + 1 < nblk))
        def _():
            drain(srcc.at[pl.ds(t0, KB)], sblk.at[slot2_b], isem.at[0])
            drain(srcc.at[pl.ds(t0, KB)], dblk.at[slot2_b], isem.at[1])

            @pl.when(slot2_b == 0)
            def _():
                _transform_idx(sblk, 0, KB, idx_off)

            @pl.when(slot2_b == 1)
            def _():
                _transform_idx(sblk, 1, KB, idx_off)

        last = rg == gpb - 1
        for k in range(4):
            q = (1 - p) * 4 + k

            @pl.when(g >= 1)
            def _():
                drain(rows_dummy, rowsb.at[q], ssem.at[q])
                if deg is not None:
                    drain(zdeg.at[pl.ds(0, CH)], ones, dsem.at[q])

            @pl.when(g + 1 < ngroups)
            def _():
                slot2 = jnp.where(last, (b + 1) & 1, b & 1)
                row2 = jnp.where(last, k, (rg + 1) * 4 + k)
                pltpu.async_copy(table.at[sblk.at[slot2, row2]],
                                 rowsb.at[q], gsem.at[q])

        @pl.when((rg == 0) & (b + 1 < nblk))
        def _():
            slot = (b + 1) & 1
            off = t0 + (b + 1) * KB
            pltpu.async_copy(srcc.at[pl.ds(off, KB)], sblk.at[slot],
                             isem.at[0])
            pltpu.async_copy(dstc.at[pl.ds(off, KB)], dblk.at[slot],
                             isem.at[1])

        b2 = b + last
        rg2 = jnp.where(last, 0, rg + 1)
        return b2, rg2

    def body(i, carry):
        b, rg = carry
        g0 = i * 2
        b, rg = group(g0, 0, b, rg)
        b, rg = group(g0 + 1, 1, b, rg)
        return b, rg

    lax.fori_loop(0, ngroups // 2, body, (jnp.int32(0), jnp.int32(0)))

    pf = (ngroups - 1) % 2
    for k in range(4):
        q = pf * 4 + k
        drain(rows_dummy, rowsb.at[q], ssem.at[q])
        if deg is not None:
            drain(zdeg.at[pl.ds(0, CH)], ones, dsem.at[q])


@functools.partial(
    pl.kernel,
    out_type=(
        jax.ShapeDtypeStruct((NPAD2, 2, 16), jnp.float32),
        jax.ShapeDtypeStruct((NPAD2,), jnp.float32),
    ),
    mesh=_mesh,
    scratch_types=(
        pltpu.VMEM_SHARED((NPAD, 1, 16), jnp.float32),
        pltpu.VMEM_SHARED((NPAD,), jnp.float32),
        pltpu.VMEM((2, KB, CH), jnp.int32),
        pltpu.VMEM((2, KB, CH), jnp.int32),
        pltpu.VMEM((8, CH, 1, 16), jnp.float32),
        pltpu.VMEM((CH,), jnp.float32),
        pltpu.VMEM((DZ,), jnp.float32),
        pltpu.SemaphoreType.DMA((8,)),
        pltpu.SemaphoreType.DMA((8,)),
        pltpu.SemaphoreType.DMA((8,)),
        pltpu.SemaphoreType.DMA((2,)),
    ),
    compiler_params=_sc_params,
)
def _sc_layer1(z1t, srcc, dstc, zrows, zdeg,
               outS, outD,
               acc, degs, sblk, dblk, rowsb, ones, degv,
               gsem, ssem, dsem, isem):
    c = lax.axis_index("c")
    s = lax.axis_index("s")
    r0 = s * RPT

    pltpu.sync_copy(zrows, acc.at[pl.ds(r0, RPT)])
    pltpu.sync_copy(zdeg, degv)
    for i in range(2):
        pltpu.sync_copy(degv, degs.at[pl.ds(r0 + i * DZ, DZ)])

    for i in range(CH // 16):
        ones[pl.ds(i * 16, 16)] = jnp.ones((16,), jnp.float32)

    plsc.subcore_barrier()

    t0 = s * L1_PER_TILE
    _edge_pipeline(z1t, srcc, dstc, acc, t0, L1_PER_TILE, c,
                   sblk, dblk, rowsb, gsem, ssem, isem,
                   deg=degs, ones=ones, dsem=dsem, zdeg=zdeg)

    plsc.subcore_barrier()

    pltpu.sync_copy(acc.at[pl.ds(r0, RPT)],
                    outS.at[pl.ds(r0, RPT), pl.ds(c, 1)])

    @pl.when(c == 0)
    def _():
        for i in range(2):
            pltpu.sync_copy(degs.at[pl.ds(r0 + i * DZ, DZ)], degv)
            pltpu.sync_copy(degv, outD.at[pl.ds(r0 + i * DZ, DZ)])


@functools.partial(
    pl.kernel,
    out_type=jax.ShapeDtypeStruct((NPAD2, 2, 16), jnp.float32),
    mesh=_mesh,
    scratch_types=(
        pltpu.VMEM_SHARED((NPAD, 1, 16), jnp.float32),
        pltpu.VMEM((2, KB, CH), jnp.int32),
        pltpu.VMEM((2, KB, CH), jnp.int32),
        pltpu.VMEM((8, CH, 1, 16), jnp.float32),
        pltpu.SemaphoreType.DMA((8,)),
        pltpu.SemaphoreType.DMA((8,)),
        pltpu.SemaphoreType.DMA((2,)),
    ),
    compiler_params=_sc_params,
)
def _sc_layer2(z2t, srcc, dstc, zrows,
               outP,
               acc, sblk, dblk, rowsb, gsem, ssem, isem):
    c = lax.axis_index("c")
    s = lax.axis_index("s")
    r0 = s * RPT

    pltpu.sync_copy(zrows, acc.at[pl.ds(r0, RPT)])
    plsc.subcore_barrier()

    t0 = c * (NCHUNK // NC) + s * L2_PER_TILE
    _edge_pipeline(z2t, srcc, dstc, acc, t0, L2_PER_TILE, jnp.int32(0),
                   sblk, dblk, rowsb, gsem, ssem, isem)

    plsc.subcore_barrier()

    pltpu.sync_copy(acc.at[pl.ds(r0, RPT)],
                    outP.at[pl.ds(r0, RPT), pl.ds(c, 1)])


NB = 4096
TCG = 25
WR = NB // 4


def _pad600(v):
    return jnp.concatenate([v, jnp.zeros((600, 128), jnp.float32)], axis=0)


def _pre1_body(xv_ref, w_ref, o_ref):
    o_ref[...] = _pad600(jnp.reshape(xv_ref[...] @ w_ref[...],
                                     (2 * 12500, 128)))


def _pre2_body(xv_ref, w_ref, b_ref, o_ref):
    o_ref[...] = _pad600(jnp.reshape(xv_ref[...] @ w_ref[...] + b_ref[...],
                                     (2 * 12500, 128)))


def _tc1_body(s_ref, xr_ref, dw_ref, wz_ref, b2_ref, o_ref):
    recipw = 1.0 / jnp.maximum(dw_ref[...], 1.0)
    h = jnp.maximum(recipw * s_ref[...] + xr_ref[...], 0.0)
    o_ref[...] = h @ wz_ref[...] + b2_ref[...]


def _tc2_body(p_ref, zhb_ref, dw_ref, g_ref, m01_ref, mh_ref, o_ref):
    recip16 = (1.0 / jnp.maximum(dw_ref[...], 1.0)) @ g_ref[...]
    psum = p_ref[...] @ m01_ref[...]
    hb = zhb_ref[...] @ mh_ref[...]
    o_ref[...] = psum * recip16 + hb


def kernel(x, edge_index, W_l1, W_r1, b1, W_l2, W_r2, b2):
    src = edge_index[0]
    dst = edge_index[1]

    pad = EPAD - E
    ar = jnp.arange(pad, dtype=jnp.int32)
    src_p = jnp.concatenate([src, (ar * 97) % N]).reshape(NCHUNK, CH)
    dst_p = jnp.concatenate([dst, N + (ar % 8)]).reshape(NCHUNK, CH)

    zrows = jnp.zeros((RPT, 1, 16), jnp.float32)
    zdeg = jnp.zeros((DZ,), jnp.float32)

    xv = x.reshape(12500, 256)
    eye8 = jnp.eye(8, dtype=jnp.float32)
    eye4 = jnp.eye(4, dtype=jnp.float32)
    wl1k = jnp.kron(eye8, W_l1)
    wr1k = jnp.kron(eye8, W_r1)
    b1t8 = jnp.tile(b1, 8).reshape(1, 256)
    wz = jnp.kron(eye4, jnp.concatenate([W_l2, W_r2], axis=1))
    b2t = jnp.tile(jnp.concatenate([jnp.zeros(16, jnp.float32), b2]),
                   4).reshape(1, 128)
    i16 = np.eye(16, dtype=np.float32)
    z16 = np.zeros((16, 16), np.float32)
    e4 = np.eye(4, dtype=np.float32)
    m01 = jnp.asarray(np.kron(e4, np.vstack([i16, i16])))
    mh = jnp.asarray(np.kron(e4, np.vstack([z16, i16])))
    g16 = jnp.asarray(np.kron(e4, np.vstack([i16, z16])))

    full = lambda r, c: pl.BlockSpec((r, c), lambda i: (0, 0))
    wideb = lambda: pl.BlockSpec((WR, 128), lambda i: (i, 0))

    z1w = pl.pallas_call(
        _pre1_body,
        in_specs=[pl.BlockSpec((12500, 256), lambda: (0, 0)),
                  pl.BlockSpec((256, 256), lambda: (0, 0))],
        out_specs=pl.BlockSpec((25600, 128), lambda: (0, 0)),
        out_shape=jax.ShapeDtypeStruct((25600, 128), jnp.float32),
    )(xv, wl1k)
    xrw = pl.pallas_call(
        _pre2_body,
        in_specs=[pl.BlockSpec((12500, 256), lambda: (0, 0)),
                  pl.BlockSpec((256, 256), lambda: (0, 0)),
                  pl.BlockSpec((1, 256), lambda: (0, 0))],
        out_specs=pl.BlockSpec((25600, 128), lambda: (0, 0)),
        out_shape=jax.ShapeDtypeStruct((25600, 128), jnp.float32),
    )(xv, wr1k, b1t8)

    outS, deg = _sc_layer1(z1w.reshape(TN, 1, 16), src_p, dst_p, zrows, zdeg)
    degw = jnp.repeat(deg, 32).reshape(25600, 128)

    z2hb = pl.pallas_call(
        _tc1_body,
        grid=(TCG,),
        in_specs=[wideb(), wideb(), wideb(),
                  full(128, 128), full(1, 128)],
        out_specs=wideb(),
        out_shape=jax.ShapeDtypeStruct((25600, 128), jnp.float32),
    )(outS.reshape(25600, 128), xrw, degw, wz, b2t)

    outP = _sc_layer2(z2hb.reshape(TN, 1, 16), src_p, dst_p, zrows)

    outw = pl.pallas_call(
        _tc2_body,
        grid=(TCG,),
        in_specs=[wideb(), wideb(), wideb(),
                  full(128, 64), full(128, 64), full(128, 64)],
        out_specs=pl.BlockSpec((WR, 64), lambda i: (i, 0)),
        out_shape=jax.ShapeDtypeStruct((25600, 64), jnp.float32),
    )(outP.reshape(25600, 128), z2hb, degw, g16, m01, mh)

    return outw.reshape(NPAD2, 16)[:N]


# speedup vs baseline: 2.0942x; 2.0942x over previous
import functools

import jax
import jax.numpy as jnp
import numpy as np
from jax import lax
from jax.experimental import pallas as pl
from jax.experimental.pallas import tpu as pltpu
from jax.experimental.pallas import tpu_sc as plsc

N = 100000
E = 1600000
D_IN = 32
D_HID = 32
D_OUT = 16

NC = 2
NS = 16

CH = 128
NCHUNK = 12544
EPAD = NCHUNK * CH
NPAD = 100096
RPT = NPAD // NS
NPAD2 = 102400
DZ = RPT // 2
TN = 204800

L1_PER_TILE = NCHUNK // NS
L2_PER_TILE = NCHUNK // (NC * NS)
KB = 8

_mesh = plsc.VectorSubcoreMesh(
    core_axis_name="c", subcore_axis_name="s", num_cores=NC, num_subcores=NS)

_sc_params = pltpu.CompilerParams(use_tc_tiling_on_sc=False)


def _transform_idx(sblk, slot, nrows, mult_off):
    off = mult_off
    for row in range(nrows):
        for i in range(CH // 16):
            v = sblk[slot, row, pl.ds(i * 16, 16)]
            sblk[slot, row, pl.ds(i * 16, 16)] = v * 2 + off


def _edge_pipeline(table, srcc, dstc, acc, t0, nchunks, idx_off,
                   sblk, dblk, rowsb, gsem, ssem, isem,
                   deg=None, ones=None, dsem=None, zdeg=None):
    gpb = KB // 4
    nblk = nchunks // KB
    ngroups = nchunks // 4

    def drain(dummy_src, dst, sem):
        pltpu.make_async_copy(dummy_src, dst, sem).wait()

    rows_dummy = table.at[pl.ds(0, CH)]

    pltpu.sync_copy(srcc.at[pl.ds(t0, KB)], sblk.at[0])
    pltpu.sync_copy(dstc.at[pl.ds(t0, KB)], dblk.at[0])
    _transform_idx(sblk, 0, KB, idx_off)
    for k in range(4):
        pltpu.async_copy(table.at[sblk.at[0, k]], rowsb.at[k], gsem.at[k])

    def group(g, p, b, rg):
        for k in range(4):
            q = p * 4 + k
            drain(rows_dummy, rowsb.at[q], gsem.at[q])
            pltpu.async_copy(rowsb.at[q], acc.at[dblk.at[b & 1, rg * 4 + k]],
                             ssem.at[q], add=True)
            if deg is not None:
                pltpu.async_copy(ones, deg.at[dblk.at[b & 1, rg * 4 + k]],
                                 dsem.at[q], add=True)

        slot2_b = (b + 1) & 1

        @pl.when((rg == gpb - 1) & (b + 1 < nblk))
        def _():
            drain(srcc.at[pl.ds(t0, KB)], sblk.at[slot2_b], isem.at[0])
            drain(srcc.at[pl.ds(t0, KB)], dblk.at[slot2_b], isem.at[1])

            @pl.when(slot2_b == 0)
            def _():
                _transform_idx(sblk, 0, KB, idx_off)

            @pl.when(slot2_b == 1)
            def _():
                _transform_idx(sblk, 1, KB, idx_off)

        last = rg == gpb - 1
        for k in range(4):
            q = (1 - p) * 4 + k

            @pl.when(g >= 1)
            def _():
                drain(rows_dummy, rowsb.at[q], ssem.at[q])
                if deg is not None:
                    drain(zdeg.at[pl.ds(0, CH)], ones, dsem.at[q])

            @pl.when(g + 1 < ngroups)
            def _():
                slot2 = jnp.where(last, (b + 1) & 1, b & 1)
                row2 = jnp.where(last, k, (rg + 1) * 4 + k)
                pltpu.async_copy(table.at[sblk.at[slot2, row2]],
                                 rowsb.at[q], gsem.at[q])

        @pl.when((rg == 0) & (b + 1 < nblk))
        def _():
            slot = (b + 1) & 1
            off = t0 + (b + 1) * KB
            pltpu.async_copy(srcc.at[pl.ds(off, KB)], sblk.at[slot],
                             isem.at[0])
            pltpu.async_copy(dstc.at[pl.ds(off, KB)], dblk.at[slot],
                             isem.at[1])

        b2 = b + last
        rg2 = jnp.where(last, 0, rg + 1)
        return b2, rg2

    def body(i, carry):
        b, rg = carry
        g0 = i * 2
        b, rg = group(g0, 0, b, rg)
        b, rg = group(g0 + 1, 1, b, rg)
        return b, rg

    lax.fori_loop(0, ngroups // 2, body, (jnp.int32(0), jnp.int32(0)))

    pf = (ngroups - 1) % 2
    for k in range(4):
        q = pf * 4 + k
        drain(rows_dummy, rowsb.at[q], ssem.at[q])
        if deg is not None:
            drain(zdeg.at[pl.ds(0, CH)], ones, dsem.at[q])


@functools.partial(
    pl.kernel,
    out_type=(
        jax.ShapeDtypeStruct((NPAD2, 32), jnp.float32),
        jax.ShapeDtypeStruct((NPAD2,), jnp.float32),
    ),
    mesh=_mesh,
    scratch_types=(
        pltpu.VMEM_SHARED((NPAD, 16), jnp.float32),
        pltpu.VMEM_SHARED((NPAD,), jnp.float32),
        pltpu.VMEM((2, KB, CH), jnp.int32),
        pltpu.VMEM((2, KB, CH), jnp.int32),
        pltpu.VMEM((8, CH, 16), jnp.float32),
        pltpu.VMEM((CH,), jnp.float32),
        pltpu.VMEM((DZ,), jnp.float32),
        pltpu.SemaphoreType.DMA((8,)),
        pltpu.SemaphoreType.DMA((8,)),
        pltpu.SemaphoreType.DMA((8,)),
        pltpu.SemaphoreType.DMA((2,)),
    ),
    compiler_params=_sc_params,
)
def _sc_layer1(z1t, srcc, dstc, zrows, zdeg,
               outS, outD,
               acc, degs, sblk, dblk, rowsb, ones, degv,
               gsem, ssem, dsem, isem):
    c = lax.axis_index("c")
    s = lax.axis_index("s")
    r0 = s * RPT

    pltpu.sync_copy(zrows, acc.at[pl.ds(r0, RPT)])
    pltpu.sync_copy(zdeg, degv)
    for i in range(2):
        pltpu.sync_copy(degv, degs.at[pl.ds(r0 + i * DZ, DZ)])

    for i in range(CH // 16):
        ones[pl.ds(i * 16, 16)] = jnp.ones((16,), jnp.float32)

    plsc.subcore_barrier()

    t0 = s * L1_PER_TILE
    _edge_pipeline(z1t, srcc, dstc, acc, t0, L1_PER_TILE, c,
                   sblk, dblk, rowsb, gsem, ssem, isem,
                   deg=degs, ones=ones, dsem=dsem, zdeg=zdeg)

    plsc.subcore_barrier()

    pltpu.sync_copy(acc.at[pl.ds(r0, RPT)],
                    outS.at[pl.ds(r0, RPT), pl.ds(16 * c, 16)])

    @pl.when(c == 0)
    def _():
        for i in range(2):
            pltpu.sync_copy(degs.at[pl.ds(r0 + i * DZ, DZ)], degv)
            pltpu.sync_copy(degv, outD.at[pl.ds(r0 + i * DZ, DZ)])


@functools.partial(
    pl.kernel,
    out_type=jax.ShapeDtypeStruct((NPAD2, 32), jnp.float32),
    mesh=_mesh,
    scratch_types=(
        pltpu.VMEM_SHARED((NPAD, 16), jnp.float32),
        pltpu.VMEM((2, KB, CH), jnp.int32),
        pltpu.VMEM((2, KB, CH), jnp.int32),
        pltpu.VMEM((8, CH, 16), jnp.float32),
        pltpu.SemaphoreType.DMA((8,)),
        pltpu.SemaphoreType.DMA((8,)),
        pltpu.SemaphoreType.DMA((2,)),
    ),
    compiler_params=_sc_params,
)
def _sc_layer2(z2t, srcc, dstc, zrows,
               outP,
               acc, sblk, dblk, rowsb, gsem, ssem, isem):
    c = lax.axis_index("c")
    s = lax.axis_index("s")
    r0 = s * RPT

    pltpu.sync_copy(zrows, acc.at[pl.ds(r0, RPT)])
    plsc.subcore_barrier()

    t0 = c * (NCHUNK // NC) + s * L2_PER_TILE
    _edge_pipeline(z2t, srcc, dstc, acc, t0, L2_PER_TILE, jnp.int32(0),
                   sblk, dblk, rowsb, gsem, ssem, isem)

    plsc.subcore_barrier()

    pltpu.sync_copy(acc.at[pl.ds(r0, RPT)],
                    outP.at[pl.ds(r0, RPT), pl.ds(16 * c, 16)])


NB = 4096
TCG = 25
WR = NB // 4


def _pad600(v):
    return jnp.concatenate([v, jnp.zeros((600, 128), jnp.float32)], axis=0)


def _pre1_body(xv_ref, w_ref, o_ref):
    o_ref[...] = _pad600(jnp.reshape(xv_ref[...] @ w_ref[...],
                                     (2 * 12500, 128)))


def _pre2_body(xv_ref, w_ref, b_ref, o_ref):
    o_ref[...] = _pad600(jnp.reshape(xv_ref[...] @ w_ref[...] + b_ref[...],
                                     (2 * 12500, 128)))


def _tc1_body(s_ref, xr_ref, dw_ref, wz_ref, b2_ref, o_ref):
    recipw = 1.0 / jnp.maximum(dw_ref[...], 1.0)
    h = jnp.maximum(recipw * s_ref[...] + xr_ref[...], 0.0)
    o_ref[...] = h @ wz_ref[...] + b2_ref[...]


def _tc2_body(p_ref, zhb_ref, dw_ref, g_ref, m01_ref, mh_ref, o_ref):
    recip16 = (1.0 / jnp.maximum(dw_ref[...], 1.0)) @ g_ref[...]
    psum = p_ref[...] @ m01_ref[...]
    hb = zhb_ref[...] @ mh_ref[...]
    o_ref[...] = psum * recip16 + hb


def kernel(x, edge_index, W_l1, W_r1, b1, W_l2, W_r2, b2):
    src = edge_index[0]
    dst = edge_index[1]

    pad = EPAD - E
    ar = jnp.arange(pad, dtype=jnp.int32)
    src_p = jnp.concatenate([src, (ar * 97) % N]).reshape(NCHUNK, CH)
    dst_p = jnp.concatenate([dst, N + (ar % 8)]).reshape(NCHUNK, CH)

    zrows = jnp.zeros((RPT, 16), jnp.float32)
    zdeg = jnp.zeros((DZ,), jnp.float32)

    xv = x.reshape(12500, 256)
    eye8 = jnp.eye(8, dtype=jnp.float32)
    eye4 = jnp.eye(4, dtype=jnp.float32)
    wl1k = jnp.kron(eye8, W_l1)
    wr1k = jnp.kron(eye8, W_r1)
    b1t8 = jnp.tile(b1, 8).reshape(1, 256)
    wz = jnp.kron(eye4, jnp.concatenate([W_l2, W_r2], axis=1))
    b2t = jnp.tile(jnp.concatenate([jnp.zeros(16, jnp.float32), b2]),
                   4).reshape(1, 128)
    i16 = np.eye(16, dtype=np.float32)
    z16 = np.zeros((16, 16), np.float32)
    e4 = np.eye(4, dtype=np.float32)
    m01 = jnp.asarray(np.kron(e4, np.vstack([i16, i16])))
    mh = jnp.asarray(np.kron(e4, np.vstack([z16, i16])))
    g16 = jnp.asarray(np.kron(e4, np.vstack([i16, z16])))

    full = lambda r, c: pl.BlockSpec((r, c), lambda i: (0, 0))
    wideb = lambda: pl.BlockSpec((WR, 128), lambda i: (i, 0))

    z1w = pl.pallas_call(
        _pre1_body,
        in_specs=[pl.BlockSpec((12500, 256), lambda: (0, 0)),
                  pl.BlockSpec((256, 256), lambda: (0, 0))],
        out_specs=pl.BlockSpec((25600, 128), lambda: (0, 0)),
        out_shape=jax.ShapeDtypeStruct((25600, 128), jnp.float32),
    )(xv, wl1k)
    xrw = pl.pallas_call(
        _pre2_body,
        in_specs=[pl.BlockSpec((12500, 256), lambda: (0, 0)),
                  pl.BlockSpec((256, 256), lambda: (0, 0)),
                  pl.BlockSpec((1, 256), lambda: (0, 0))],
        out_specs=pl.BlockSpec((25600, 128), lambda: (0, 0)),
        out_shape=jax.ShapeDtypeStruct((25600, 128), jnp.float32),
    )(xv, wr1k, b1t8)

    outS, deg = _sc_layer1(z1w.reshape(TN, 16), src_p, dst_p, zrows, zdeg)
    degw = jnp.repeat(deg, 32).reshape(25600, 128)

    z2hb = pl.pallas_call(
        _tc1_body,
        grid=(TCG,),
        in_specs=[wideb(), wideb(), wideb(),
                  full(128, 128), full(1, 128)],
        out_specs=wideb(),
        out_shape=jax.ShapeDtypeStruct((25600, 128), jnp.float32),
    )(outS.reshape(25600, 128), xrw, degw, wz, b2t)

    outP = _sc_layer2(z2hb.reshape(TN, 16), src_p, dst_p, zrows)

    outw = pl.pallas_call(
        _tc2_body,
        grid=(TCG,),
        in_specs=[wideb(), wideb(), wideb(),
                  full(128, 64), full(128, 64), full(128, 64)],
        out_specs=pl.BlockSpec((WR, 64), lambda i: (i, 0)),
        out_shape=jax.ShapeDtypeStruct((25600, 64), jnp.float32),
    )(outP.reshape(25600, 128), z2hb, degw, g16, m01, mh)

    return outw.reshape(NPAD2, 16)[:N]
